# baseline (speedup 1.0000x reference)
import functools

import jax
import jax.numpy as jnp
from jax import lax
from jax.experimental import pallas as pl
from jax.experimental.pallas import tpu as pltpu
from jax.experimental.pallas import tpu_sc as plsc

_B = 1024
_D = 64
_IN = 512
_H = 256
_C = 1000
_M = 100000
_K = 8
_DECAY = 0.01

_BB = 64
_NB = _B // _BB
_TM = 8192
_MT = 13
_MP = _MT * _TM
_GW = 128
_GPT = _TM // _GW
_G = _MT * _GPT
_NEG = -1e30


def _enc_body(x_ref, encw_ref, encb_ref, z_out, zn_out):
    z = jnp.dot(x_ref[...], encw_ref[...],
                preferred_element_type=jnp.float32) + encb_ref[...][None, :]
    z_out[...] = z
    nrm = jnp.sqrt(jnp.sum(z * z, axis=1, keepdims=True))
    zn_out[...] = z / jnp.maximum(nrm, 1e-8)


_ENC_CALL_KW = dict(
    out_shape=[
        jax.ShapeDtypeStruct((_B, _D), jnp.float32),
        jax.ShapeDtypeStruct((_B, _D), jnp.float32),
    ],
)


def _prescale_body(c_ref, r0_ref, tau_ref, cs_out):
    c = c_ref[...]
    cn = c / jnp.maximum(
        jnp.sqrt(jnp.sum(c * c, axis=1, keepdims=True)), 1e-8)
    sal = r0_ref[...] * jnp.exp(-_DECAY * tau_ref[...])
    cs_out[...] = cn * sal[:, None]


_PRESCALE_CALL_KW = dict(
    grid=(_MT,),
    in_specs=[
        pl.BlockSpec((_TM, _D), lambda m: (m, 0)),
        pl.BlockSpec((_TM,), lambda m: (m,)),
        pl.BlockSpec((_TM,), lambda m: (m,)),
    ],
    out_specs=pl.BlockSpec((_TM, _D), lambda m: (m, 0)),
    out_shape=jax.ShapeDtypeStruct((_M, _D), jnp.float32),
)


def _topk_body(zn_in, cs_ref,
               topv_out, topi_out, gidx_out,
               s3_ref, gm_ref, gid_ref, cand_ref):
    m = pl.program_id(1)
    zn = zn_in[...]

    score = lax.dot_general(zn, cs_ref[...],
                            (((1,), (1,)), ((), ())),
                            preferred_element_type=jnp.float32)
    col = m * _TM + lax.broadcasted_iota(jnp.int32, (1, _TM), 1)
    score = jnp.where(col < _M, score, _NEG)

    s3_ref[:, pl.ds(m * _GPT, _GPT), :] = score.reshape(_BB, _GPT, _GW)
    gm_ref[:, m, :] = jnp.max(score.reshape(_BB, _GPT, _GW), axis=2)

    @pl.when(m == _MT - 1)
    def _select():
        work = gm_ref[...].reshape(_BB, _G)
        giota = lax.broadcasted_iota(jnp.int32, (_BB, _G), 1)
        gids = []
        for _ in range(_K):
            mx = jnp.max(work, axis=1, keepdims=True)
            pos = jnp.min(jnp.where(work == mx, giota, _G), axis=1)
            gids.append(pos)
            work = jnp.where(giota == pos[:, None], _NEG, work)
        gid_ref[...] = jnp.stack(gids, axis=1).astype(jnp.int32)

        def _gather_one(b, carry):
            for k in range(_K):
                g = gid_ref[b, k]
                g8 = pl.multiple_of((g // 8) * 8, 8)
                chunk = s3_ref[b, pl.ds(g8, 8), :]
                rsel = lax.broadcasted_iota(jnp.int32, (8, _GW), 0) == (g - g8)
                cand_ref[b, k, :] = jnp.max(
                    jnp.where(rsel, chunk, _NEG), axis=0)
            return carry
        lax.fori_loop(0, _BB, _gather_one, 0, unroll=False)

        cand = cand_ref[...].reshape(_BB, _K * _GW)
        ciota = lax.broadcasted_iota(jnp.int32, (_BB, _K * _GW), 1)
        gid_v = gid_ref[...]
        vals, idxs = [], []
        for _ in range(_K):
            mx = jnp.max(cand, axis=1, keepdims=True)
            pos = jnp.min(jnp.where(cand == mx, ciota, _K * _GW), axis=1)
            cand = jnp.where(ciota == pos[:, None], _NEG, cand)
            slot = pos // _GW
            onehot = lax.broadcasted_iota(jnp.int32, (_BB, _K), 1) == slot[:, None]
            g_of = jnp.sum(jnp.where(onehot, gid_v, 0), axis=1)
            vals.append(mx[:, 0])
            idxs.append(g_of * _GW + (pos % _GW))
        topv_out[...] = jnp.stack(vals, axis=1)
        topi = jnp.stack(idxs, axis=1).astype(jnp.int32)
        topi_out[...] = topi
        gidx_out[...] = topi // 2


_TOPK_CALL_KW = dict(
    grid=(_NB, _MT),
    in_specs=[
        pl.BlockSpec((_BB, _D), lambda b, m: (b, 0)),
        pl.BlockSpec((_TM, _D), lambda b, m: (m, 0)),
    ],
    out_specs=[
        pl.BlockSpec((_BB, _K), lambda b, m: (b, 0)),
        pl.BlockSpec((_BB, _K), lambda b, m: (b, 0)),
        pl.BlockSpec((_BB, _K), lambda b, m: (b, 0)),
    ],
    out_shape=[
        jax.ShapeDtypeStruct((_B, _K), jnp.float32),
        jax.ShapeDtypeStruct((_B, _K), jnp.int32),
        jax.ShapeDtypeStruct((_B, _K), jnp.int32),
    ],
    scratch_shapes=[
        pltpu.VMEM((_BB, _G, _GW), jnp.float32),
        pltpu.VMEM((_BB, _MT, _GPT), jnp.float32),
        pltpu.VMEM((_BB, _K), jnp.int32),
        pltpu.VMEM((_BB, _K, _GW), jnp.float32),
    ],
    compiler_params=pltpu.CompilerParams(
        dimension_semantics=("arbitrary", "arbitrary")),
)


def _gather_rows(table, idx):
    info = plsc.get_sparse_core_info()
    nc, ns = info.num_cores, info.num_subcores
    nw = nc * ns
    n = _B * _K
    bpw = n // nw
    mesh = plsc.VectorSubcoreMesh(core_axis_name="c", subcore_axis_name="s")

    @functools.partial(
        pl.kernel, mesh=mesh,
        out_type=jax.ShapeDtypeStruct((n, 2 * _D), jnp.float32),
        scratch_types=[
            pltpu.VMEM((bpw,), jnp.int32),
            pltpu.VMEM((bpw, 2 * _D), jnp.float32),
            pltpu.SemaphoreType.DMA,
        ],
    )
    def _k(table_hbm, idx_hbm, out_hbm, idx_v, rows_v, sem):
        wid = lax.axis_index("s") * nc + lax.axis_index("c")
        base = wid * bpw
        pltpu.sync_copy(idx_hbm.at[pl.ds(base, bpw)], idx_v)
        pltpu.async_copy(table_hbm.at[idx_v], rows_v, sem).wait()
        pltpu.sync_copy(rows_v, out_hbm.at[pl.ds(base, bpw)])

    return _k(table, idx)


def _dec_body(z_ref, topv_ref, topi_ref, rows_ref, w1_ref, b1_ref, w2_ref,
              b2_ref, out_ref):
    v = topv_ref[...]
    e = jnp.exp(v - jnp.max(v, axis=1, keepdims=True))
    w = e / jnp.sum(e, axis=1, keepdims=True)
    rows = rows_ref[...]
    par = topi_ref[...] % 2
    r = jnp.zeros((_B, _D), jnp.float32)
    for k in range(_K):
        lo = rows[:, k * 2 * _D:k * 2 * _D + _D]
        hi = rows[:, k * 2 * _D + _D:(k + 1) * 2 * _D]
        rk = jnp.where(par[:, k:k + 1] == 0, lo, hi)
        r = r + rk * w[:, k:k + 1]
    h = (jnp.dot(z_ref[...], w1_ref[0:_D, :],
                 preferred_element_type=jnp.float32)
         + jnp.dot(r, w1_ref[_D:2 * _D, :],
                   preferred_element_type=jnp.float32)
         + b1_ref[...][None, :])
    h = jnp.maximum(h, 0.0)
    out_ref[...] = (jnp.dot(h, w2_ref[...], preferred_element_type=jnp.float32)
                    + b2_ref[...][None, :])


_DEC_CALL_KW = dict(
    out_shape=jax.ShapeDtypeStruct((_B, _C), jnp.float32),
)


def kernel(x, enc_W, enc_b, dec_W1, dec_b1, dec_W2, dec_b2,
           z_buffer, c_buffer, r0_buffer, tau_buffer):
    z, zn = pl.pallas_call(_enc_body, **_ENC_CALL_KW)(x, enc_W, enc_b)
    cs = pl.pallas_call(_prescale_body, **_PRESCALE_CALL_KW)(
        c_buffer, r0_buffer, tau_buffer)
    topv, topi, gidx = pl.pallas_call(_topk_body, **_TOPK_CALL_KW)(
        zn, cs)
    rows = _gather_rows(z_buffer.reshape(_M // 2, 2 * _D),
                        gidx.reshape(_B * _K))
    logits = pl.pallas_call(_dec_body, **_DEC_CALL_KW)(
        z, topv, topi, rows.reshape(_B, _K * 2 * _D),
        dec_W1, dec_b1, dec_W2, dec_b2)
    return logits


# speedup vs baseline: 1.0050x; 1.0050x over previous
import functools

import jax
import jax.numpy as jnp
from jax import lax
from jax.experimental import pallas as pl
from jax.experimental.pallas import tpu as pltpu
from jax.experimental.pallas import tpu_sc as plsc

_B = 1024
_D = 64
_IN = 512
_H = 256
_C = 1000
_M = 100000
_K = 8
_DECAY = 0.01

_BB = 64
_NB = _B // _BB
_TM = 8192
_MT = 13
_MP = _MT * _TM
_GW = 128
_GPT = _TM // _GW
_G = _MT * _GPT
_NEG = -1e30


def _enc_body(x_ref, encw_ref, encb_ref, z_out, zn_out):
    z = jnp.dot(x_ref[...], encw_ref[...],
                preferred_element_type=jnp.float32) + encb_ref[...][None, :]
    z_out[...] = z
    nrm = jnp.sqrt(jnp.sum(z * z, axis=1, keepdims=True))
    zn_out[...] = z / jnp.maximum(nrm, 1e-8)


_ENC_CALL_KW = dict(
    out_shape=[
        jax.ShapeDtypeStruct((_B, _D), jnp.float32),
        jax.ShapeDtypeStruct((_B, _D), jnp.float32),
    ],
)


def _prescale_body(c_ref, r0_ref, tau_ref, cn_out, sal_out):
    c = c_ref[...]
    cn_out[...] = c / jnp.maximum(
        jnp.sqrt(jnp.sum(c * c, axis=1, keepdims=True)), 1e-8)
    sal_out[...] = r0_ref[...] * jnp.exp(-_DECAY * tau_ref[...])


_PRESCALE_CALL_KW = dict(
    grid=(_MT,),
    in_specs=[
        pl.BlockSpec((_TM, _D), lambda m: (m, 0)),
        pl.BlockSpec((_TM,), lambda m: (m,)),
        pl.BlockSpec((_TM,), lambda m: (m,)),
    ],
    out_specs=[
        pl.BlockSpec((_TM, _D), lambda m: (m, 0)),
        pl.BlockSpec((_TM,), lambda m: (m,)),
    ],
    out_shape=[
        jax.ShapeDtypeStruct((_M, _D), jnp.float32),
        jax.ShapeDtypeStruct((_M,), jnp.float32),
    ],
)


def _topk_body(zn_in, cn_ref, sal_ref,
               topv_out, topi_out, gidx_out,
               s3_ref, gm_ref, gid_ref, cand_ref):
    m = pl.program_id(1)
    zn = zn_in[...]

    score = lax.dot_general(zn, cn_ref[...],
                            (((1,), (1,)), ((), ())),
                            preferred_element_type=jnp.float32)
    score = score * sal_ref[...][None, :]
    col = m * _TM + lax.broadcasted_iota(jnp.int32, (1, _TM), 1)
    score = jnp.where(col < _M, score, _NEG)

    s3_ref[:, pl.ds(m * _GPT, _GPT), :] = score.reshape(_BB, _GPT, _GW)
    gm_ref[:, m, :] = jnp.max(score.reshape(_BB, _GPT, _GW), axis=2)

    @pl.when(m == _MT - 1)
    def _select():
        work = gm_ref[...].reshape(_BB, _G)
        giota = lax.broadcasted_iota(jnp.int32, (_BB, _G), 1)
        gids = []
        for _ in range(_K):
            mx = jnp.max(work, axis=1, keepdims=True)
            pos = jnp.min(jnp.where(work == mx, giota, _G), axis=1)
            gids.append(pos)
            work = jnp.where(giota == pos[:, None], _NEG, work)
        gid_ref[...] = jnp.stack(gids, axis=1).astype(jnp.int32)

        def _gather_one(b, carry):
            for k in range(_K):
                g = gid_ref[b, k]
                g8 = pl.multiple_of((g // 8) * 8, 8)
                chunk = s3_ref[b, pl.ds(g8, 8), :]
                rsel = lax.broadcasted_iota(jnp.int32, (8, _GW), 0) == (g - g8)
                cand_ref[b, k, :] = jnp.max(
                    jnp.where(rsel, chunk, _NEG), axis=0)
            return carry
        lax.fori_loop(0, _BB, _gather_one, 0, unroll=False)

        cand = cand_ref[...].reshape(_BB, _K * _GW)
        ciota = lax.broadcasted_iota(jnp.int32, (_BB, _K * _GW), 1)
        gid_v = gid_ref[...]
        vals, idxs = [], []
        for _ in range(_K):
            mx = jnp.max(cand, axis=1, keepdims=True)
            pos = jnp.min(jnp.where(cand == mx, ciota, _K * _GW), axis=1)
            cand = jnp.where(ciota == pos[:, None], _NEG, cand)
            slot = pos // _GW
            onehot = lax.broadcasted_iota(jnp.int32, (_BB, _K), 1) == slot[:, None]
            g_of = jnp.sum(jnp.where(onehot, gid_v, 0), axis=1)
            vals.append(mx[:, 0])
            idxs.append(g_of * _GW + (pos % _GW))
        topv_out[...] = jnp.stack(vals, axis=1)
        topi = jnp.stack(idxs, axis=1).astype(jnp.int32)
        topi_out[...] = topi
        gidx_out[...] = topi // 2


_TOPK_CALL_KW = dict(
    grid=(_NB, _MT),
    in_specs=[
        pl.BlockSpec((_BB, _D), lambda b, m: (b, 0)),
        pl.BlockSpec((_TM, _D), lambda b, m: (m, 0)),
        pl.BlockSpec((_TM,), lambda b, m: (m,)),
    ],
    out_specs=[
        pl.BlockSpec((_BB, _K), lambda b, m: (b, 0)),
        pl.BlockSpec((_BB, _K), lambda b, m: (b, 0)),
        pl.BlockSpec((_BB, _K), lambda b, m: (b, 0)),
    ],
    out_shape=[
        jax.ShapeDtypeStruct((_B, _K), jnp.float32),
        jax.ShapeDtypeStruct((_B, _K), jnp.int32),
        jax.ShapeDtypeStruct((_B, _K), jnp.int32),
    ],
    scratch_shapes=[
        pltpu.VMEM((_BB, _G, _GW), jnp.float32),
        pltpu.VMEM((_BB, _MT, _GPT), jnp.float32),
        pltpu.VMEM((_BB, _K), jnp.int32),
        pltpu.VMEM((_BB, _K, _GW), jnp.float32),
    ],
    compiler_params=pltpu.CompilerParams(
        dimension_semantics=("arbitrary", "arbitrary")),
)


def _gather_rows(table, idx):
    info = plsc.get_sparse_core_info()
    nc, ns = info.num_cores, info.num_subcores
    nw = nc * ns
    n = _B * _K
    bpw = n // nw
    mesh = plsc.VectorSubcoreMesh(core_axis_name="c", subcore_axis_name="s")

    @functools.partial(
        pl.kernel, mesh=mesh,
        out_type=jax.ShapeDtypeStruct((n, 2 * _D), jnp.float32),
        scratch_types=[
            pltpu.VMEM((bpw,), jnp.int32),
            pltpu.VMEM((bpw, 2 * _D), jnp.float32),
            pltpu.SemaphoreType.DMA,
        ],
    )
    def _k(table_hbm, idx_hbm, out_hbm, idx_v, rows_v, sem):
        wid = lax.axis_index("s") * nc + lax.axis_index("c")
        base = wid * bpw
        pltpu.sync_copy(idx_hbm.at[pl.ds(base, bpw)], idx_v)
        pltpu.async_copy(table_hbm.at[idx_v], rows_v, sem).wait()
        pltpu.sync_copy(rows_v, out_hbm.at[pl.ds(base, bpw)])

    return _k(table, idx)


def _dec_body(z_ref, topv_ref, topi_ref, rows_ref, w1_ref, b1_ref, w2_ref,
              b2_ref, out_ref):
    v = topv_ref[...]
    e = jnp.exp(v - jnp.max(v, axis=1, keepdims=True))
    w = e / jnp.sum(e, axis=1, keepdims=True)
    rows = rows_ref[...]
    par = topi_ref[...] % 2
    r = jnp.zeros((_B, _D), jnp.float32)
    for k in range(_K):
        lo = rows[:, k * 2 * _D:k * 2 * _D + _D]
        hi = rows[:, k * 2 * _D + _D:(k + 1) * 2 * _D]
        rk = jnp.where(par[:, k:k + 1] == 0, lo, hi)
        r = r + rk * w[:, k:k + 1]
    h = (jnp.dot(z_ref[...], w1_ref[0:_D, :],
                 preferred_element_type=jnp.float32)
         + jnp.dot(r, w1_ref[_D:2 * _D, :],
                   preferred_element_type=jnp.float32)
         + b1_ref[...][None, :])
    h = jnp.maximum(h, 0.0)
    out_ref[...] = (jnp.dot(h, w2_ref[...], preferred_element_type=jnp.float32)
                    + b2_ref[...][None, :])


_DEC_CALL_KW = dict(
    out_shape=jax.ShapeDtypeStruct((_B, _C), jnp.float32),
)


def kernel(x, enc_W, enc_b, dec_W1, dec_b1, dec_W2, dec_b2,
           z_buffer, c_buffer, r0_buffer, tau_buffer):
    z, zn = pl.pallas_call(_enc_body, **_ENC_CALL_KW)(x, enc_W, enc_b)
    cn, sal = pl.pallas_call(_prescale_body, **_PRESCALE_CALL_KW)(
        c_buffer, r0_buffer, tau_buffer)
    topv, topi, gidx = pl.pallas_call(_topk_body, **_TOPK_CALL_KW)(
        zn, cn, sal)
    rows = _gather_rows(z_buffer.reshape(_M // 2, 2 * _D),
                        gidx.reshape(_B * _K))
    logits = pl.pallas_call(_dec_body, **_DEC_CALL_KW)(
        z, topv, topi, rows.reshape(_B, _K * 2 * _D),
        dec_W1, dec_b1, dec_W2, dec_b2)
    return logits


# speedup vs baseline: 1.1688x; 1.1630x over previous
import functools

import jax
import jax.numpy as jnp
from jax import lax
from jax.experimental import pallas as pl
from jax.experimental.pallas import tpu as pltpu
from jax.experimental.pallas import tpu_sc as plsc

_B = 1024
_D = 64
_IN = 512
_H = 256
_C = 1000
_M = 100000
_K = 8
_DECAY = 0.01

_BB = 64
_NB = _B // _BB
_TM = 8192
_MT = 13
_MP = _MT * _TM
_GW = 128
_GPT = _TM // _GW
_G = _MT * _GPT
_NEG = -1e30


def _enc_body(x_ref, encw_ref, encb_ref, z_out, zn_out):
    z = jnp.dot(x_ref[...], encw_ref[...],
                preferred_element_type=jnp.float32) + encb_ref[...][None, :]
    z_out[...] = z
    nrm = jnp.sqrt(jnp.sum(z * z, axis=1, keepdims=True))
    zn_out[...] = z / jnp.maximum(nrm, 1e-8)


_ENC_CALL_KW = dict(
    out_shape=[
        jax.ShapeDtypeStruct((_B, _D), jnp.float32),
        jax.ShapeDtypeStruct((_B, _D), jnp.float32),
    ],
)


def _prescale_body(c_ref, r0_ref, tau_ref, cn_out, sal_out):
    c = c_ref[...]
    cn = c / jnp.maximum(
        jnp.sqrt(jnp.sum(c * c, axis=1, keepdims=True)), 1e-8)
    cn_out[...] = cn.astype(jnp.bfloat16)
    sal_out[...] = r0_ref[...] * jnp.exp(-_DECAY * tau_ref[...])


_PRESCALE_CALL_KW = dict(
    grid=(_MT,),
    in_specs=[
        pl.BlockSpec((_TM, _D), lambda m: (m, 0)),
        pl.BlockSpec((_TM,), lambda m: (m,)),
        pl.BlockSpec((_TM,), lambda m: (m,)),
    ],
    out_specs=[
        pl.BlockSpec((_TM, _D), lambda m: (m, 0)),
        pl.BlockSpec((_TM,), lambda m: (m,)),
    ],
    out_shape=[
        jax.ShapeDtypeStruct((_M, _D), jnp.bfloat16),
        jax.ShapeDtypeStruct((_M,), jnp.float32),
    ],
)


def _topk_body(zn_in, cn_ref, sal_ref,
               topv_out, topi_out, gidx_out,
               s3_ref, gm_ref, gid_ref, cand_ref):
    m = pl.program_id(1)
    zn = zn_in[...]

    score = lax.dot_general(zn.astype(jnp.bfloat16), cn_ref[...],
                            (((1,), (1,)), ((), ())),
                            preferred_element_type=jnp.float32)
    score = score * sal_ref[...][None, :]
    col = m * _TM + lax.broadcasted_iota(jnp.int32, (1, _TM), 1)
    score = jnp.where(col < _M, score, _NEG)

    s3_ref[:, pl.ds(m * _TM, _TM)] = score
    gm_ref[:, m, :] = jnp.max(score.reshape(_BB, _GPT, _GW), axis=2)

    @pl.when(m == _MT - 1)
    def _select():
        work = gm_ref[...].reshape(_BB, _G)
        giota = lax.broadcasted_iota(jnp.int32, (_BB, _G), 1)
        gids = []
        for _ in range(_K):
            mx = jnp.max(work, axis=1, keepdims=True)
            pos = jnp.min(jnp.where(work == mx, giota, _G), axis=1)
            gids.append(pos)
            work = jnp.where(giota == pos[:, None], _NEG, work)
        gid_ref[...] = jnp.stack(gids, axis=1).astype(jnp.int32)

        def _gather_one(b, carry):
            b8 = pl.multiple_of((b // 8) * 8, 8)
            rsel = lax.broadcasted_iota(jnp.int32, (8, _GW), 0) == (b - b8)
            for k in range(_K):
                g = gid_ref[b, k]
                goff = pl.multiple_of(g * _GW, _GW)
                chunk = s3_ref[pl.ds(b8, 8), pl.ds(goff, _GW)]
                cand_ref[b, k, :] = jnp.max(
                    jnp.where(rsel, chunk, _NEG), axis=0)
            return carry
        lax.fori_loop(0, _BB, _gather_one, 0, unroll=False)

        cand = cand_ref[...].reshape(_BB, _K * _GW)
        ciota = lax.broadcasted_iota(jnp.int32, (_BB, _K * _GW), 1)
        gid_v = gid_ref[...]
        vals, idxs = [], []
        for _ in range(_K):
            mx = jnp.max(cand, axis=1, keepdims=True)
            pos = jnp.min(jnp.where(cand == mx, ciota, _K * _GW), axis=1)
            cand = jnp.where(ciota == pos[:, None], _NEG, cand)
            slot = pos // _GW
            onehot = lax.broadcasted_iota(jnp.int32, (_BB, _K), 1) == slot[:, None]
            g_of = jnp.sum(jnp.where(onehot, gid_v, 0), axis=1)
            vals.append(mx[:, 0])
            idxs.append(g_of * _GW + (pos % _GW))
        topv_out[...] = jnp.stack(vals, axis=1)
        topi = jnp.stack(idxs, axis=1).astype(jnp.int32)
        topi_out[...] = topi
        gidx_out[...] = topi // 2


_TOPK_CALL_KW = dict(
    grid=(_NB, _MT),
    in_specs=[
        pl.BlockSpec((_BB, _D), lambda b, m: (b, 0)),
        pl.BlockSpec((_TM, _D), lambda b, m: (m, 0)),
        pl.BlockSpec((_TM,), lambda b, m: (m,)),
    ],
    out_specs=[
        pl.BlockSpec((_BB, _K), lambda b, m: (b, 0)),
        pl.BlockSpec((_BB, _K), lambda b, m: (b, 0)),
        pl.BlockSpec((_BB, _K), lambda b, m: (b, 0)),
    ],
    out_shape=[
        jax.ShapeDtypeStruct((_B, _K), jnp.float32),
        jax.ShapeDtypeStruct((_B, _K), jnp.int32),
        jax.ShapeDtypeStruct((_B, _K), jnp.int32),
    ],
    scratch_shapes=[
        pltpu.VMEM((_BB, _MP), jnp.float32),
        pltpu.VMEM((_BB, _MT, _GPT), jnp.float32),
        pltpu.VMEM((_BB, _K), jnp.int32),
        pltpu.VMEM((_BB, _K, _GW), jnp.float32),
    ],
    compiler_params=pltpu.CompilerParams(
        dimension_semantics=("arbitrary", "arbitrary")),
)


def _gather_rows(table, idx):
    info = plsc.get_sparse_core_info()
    nc, ns = info.num_cores, info.num_subcores
    nw = nc * ns
    n = _B * _K
    bpw = n // nw
    mesh = plsc.VectorSubcoreMesh(core_axis_name="c", subcore_axis_name="s")

    @functools.partial(
        pl.kernel, mesh=mesh,
        out_type=jax.ShapeDtypeStruct((n, 2 * _D), jnp.float32),
        scratch_types=[
            pltpu.VMEM((bpw,), jnp.int32),
            pltpu.VMEM((bpw, 2 * _D), jnp.float32),
            pltpu.SemaphoreType.DMA,
        ],
    )
    def _k(table_hbm, idx_hbm, out_hbm, idx_v, rows_v, sem):
        wid = lax.axis_index("s") * nc + lax.axis_index("c")
        base = wid * bpw
        pltpu.sync_copy(idx_hbm.at[pl.ds(base, bpw)], idx_v)
        pltpu.async_copy(table_hbm.at[idx_v], rows_v, sem).wait()
        pltpu.sync_copy(rows_v, out_hbm.at[pl.ds(base, bpw)])

    return _k(table, idx)


def _dec_body(z_ref, topv_ref, topi_ref, rows_ref, w1_ref, b1_ref, w2_ref,
              b2_ref, out_ref):
    v = topv_ref[...]
    e = jnp.exp(v - jnp.max(v, axis=1, keepdims=True))
    w = e / jnp.sum(e, axis=1, keepdims=True)
    rows = rows_ref[...]
    par = topi_ref[...] % 2
    r = jnp.zeros((_B, _D), jnp.float32)
    for k in range(_K):
        lo = rows[:, k * 2 * _D:k * 2 * _D + _D]
        hi = rows[:, k * 2 * _D + _D:(k + 1) * 2 * _D]
        rk = jnp.where(par[:, k:k + 1] == 0, lo, hi)
        r = r + rk * w[:, k:k + 1]
    h = (jnp.dot(z_ref[...], w1_ref[0:_D, :],
                 preferred_element_type=jnp.float32)
         + jnp.dot(r, w1_ref[_D:2 * _D, :],
                   preferred_element_type=jnp.float32)
         + b1_ref[...][None, :])
    h = jnp.maximum(h, 0.0)
    out_ref[...] = (jnp.dot(h, w2_ref[...], preferred_element_type=jnp.float32)
                    + b2_ref[...][None, :])


_DEC_CALL_KW = dict(
    out_shape=jax.ShapeDtypeStruct((_B, _C), jnp.float32),
)


def kernel(x, enc_W, enc_b, dec_W1, dec_b1, dec_W2, dec_b2,
           z_buffer, c_buffer, r0_buffer, tau_buffer):
    z, zn = pl.pallas_call(_enc_body, **_ENC_CALL_KW)(x, enc_W, enc_b)
    cn, sal = pl.pallas_call(_prescale_body, **_PRESCALE_CALL_KW)(
        c_buffer, r0_buffer, tau_buffer)
    topv, topi, gidx = pl.pallas_call(_topk_body, **_TOPK_CALL_KW)(
        zn, cn, sal)
    rows = _gather_rows(z_buffer.reshape(_M // 2, 2 * _D),
                        gidx.reshape(_B * _K))
    logits = pl.pallas_call(_dec_body, **_DEC_CALL_KW)(
        z, topv, topi, rows.reshape(_B, _K * 2 * _D),
        dec_W1, dec_b1, dec_W2, dec_b2)
    return logits
